# baseline (speedup 1.0000x reference)
import functools

import jax
import jax.numpy as jnp
from jax import lax
from jax.experimental import pallas as pl
from jax.experimental.pallas import tpu as pltpu
from jax.experimental.pallas import tpu_sc as plsc

BATCH, HIST, DIM = 4096, 200, 128
TOTAL = BATCH * HIST


@functools.partial(jax.jit, static_argnames=())
def _embed(indices_flat, weight):
    info = plsc.get_sparse_core_info()
    nc, ns = info.num_cores, info.num_subcores
    nw = nc * ns
    per_w = TOTAL // nw
    chunk = 128
    n_chunks = per_w // chunk
    n_k = n_chunks // 2

    mesh = plsc.VectorSubcoreMesh(core_axis_name="c", subcore_axis_name="s")

    @functools.partial(
        pl.kernel,
        mesh=mesh,
        out_type=jax.ShapeDtypeStruct((TOTAL, DIM), jnp.float32),
        scratch_types=(
            [pltpu.VMEM((per_w,), jnp.int32)]
            + [pltpu.VMEM((chunk, DIM), jnp.float32)] * 4
            + [pltpu.VMEM_SHARED((ns, 2, chunk, DIM), jnp.float32)]
            + [pltpu.SemaphoreType.DMA] * 10
        ),
    )
    def k(idx_hbm, table_hbm, out_hbm, idx_v, s0, s1, d0, d1, sp,
          gs0, gs1, gd0, gd1, ss0, ss1, cs0, cs1, ws0, ws1):
        srows, drows = (s0, s1), (d0, d1)
        gsem, hsem = (gs0, gs1), (gd0, gd1)
        ssem, csem, wsem = (ss0, ss1), (cs0, cs1), (ws0, ws1)
        sid = lax.axis_index("s")
        wid = sid * nc + lax.axis_index("c")
        base = wid * per_w
        pltpu.sync_copy(idx_hbm.at[pl.ds(base, per_w)], idx_v)

        def idx_slice(i):
            return idx_v.at[pl.ds(pl.multiple_of(i * chunk, 8), chunk)]

        def out_slice(i):
            return out_hbm.at[pl.ds(base + i * chunk, chunk)]

        def sg_start(b, k_):
            pltpu.async_copy(table_hbm.at[idx_slice(2 * k_)], srows[b], gsem[b])

        def sg_wait(b, k_):
            pltpu.make_async_copy(table_hbm.at[idx_slice(2 * k_)], srows[b], gsem[b]).wait()

        def swb_start(b, k_):
            pltpu.async_copy(srows[b], out_slice(2 * k_), ssem[b])

        def swb_wait(b):
            pltpu.make_async_copy(srows[b], out_slice(0), ssem[b]).wait()

        def dg_start(b, k_):
            pltpu.async_copy(table_hbm.at[idx_slice(2 * k_ + 1)], drows[b], hsem[b])

        def dg_wait(b, k_):
            pltpu.make_async_copy(table_hbm.at[idx_slice(2 * k_ + 1)], drows[b], hsem[b]).wait()

        def dcopy_start(b):
            pltpu.async_copy(drows[b], sp.at[sid, b], csem[b])

        def dcopy_wait(b):
            pltpu.make_async_copy(drows[b], sp.at[sid, b], csem[b]).wait()

        def dwb_start(b, k_):
            pltpu.async_copy(sp.at[sid, b], out_slice(2 * k_ + 1), wsem[b])

        def dwb_wait(b):
            pltpu.make_async_copy(sp.at[sid, b], out_slice(0), wsem[b]).wait()

        for b in (0, 1):
            sg_start(b, b)
            dg_start(b, b)

        for b in (0, 1):
            sg_wait(b, b)
            swb_start(b, b)
            dg_wait(b, b)
            dcopy_start(b)
            dcopy_wait(b)
            dwb_start(b, b)
            sg_start(b, b + 2)
            dg_start(b, b + 2)

        def body(g, carry):
            for b in (0, 1):
                k_ = 2 * g + b
                nxt = jnp.minimum(k_ + 2, n_k - 1)
                sg_wait(b, k_)
                swb_wait(b)
                swb_start(b, k_)
                dg_wait(b, k_)
                dwb_wait(b)
                dcopy_start(b)
                dcopy_wait(b)
                dwb_start(b, k_)
                sg_start(b, nxt)
                dg_start(b, nxt)
            return carry

        lax.fori_loop(1, n_k // 2, body, 0)

        for b in (0, 1):
            sg_wait(b, n_k - 1)
            dg_wait(b, n_k - 1)
            swb_wait(b)
            dwb_wait(b)

    return k(indices_flat, weight)


def kernel(indices, weight):
    flat = indices.reshape(-1).astype(jnp.int32)
    out = _embed(flat, weight)
    return out.reshape(BATCH, HIST, DIM)


# speedup vs baseline: 1.0029x; 1.0029x over previous
import functools

import jax
import jax.numpy as jnp
from jax import lax
from jax.experimental import pallas as pl
from jax.experimental.pallas import tpu as pltpu
from jax.experimental.pallas import tpu_sc as plsc

BATCH, HIST, DIM = 4096, 200, 128
TOTAL = BATCH * HIST


@functools.partial(jax.jit, static_argnames=())
def _embed(indices_flat, weight):
    info = plsc.get_sparse_core_info()
    nc, ns = info.num_cores, info.num_subcores
    nw = nc * ns
    per_w = TOTAL // nw
    chunk = 160
    n_chunks = per_w // chunk
    n_groups = n_chunks // 2

    mesh = plsc.VectorSubcoreMesh(core_axis_name="c", subcore_axis_name="s")

    @functools.partial(
        pl.kernel,
        mesh=mesh,
        out_type=jax.ShapeDtypeStruct((TOTAL, DIM), jnp.float32),
        scratch_types=(
            [pltpu.VMEM((per_w,), jnp.int32)]
            + [pltpu.VMEM((chunk, DIM), jnp.float32)] * 2
            + [pltpu.VMEM_SHARED((ns, 2, chunk, DIM), jnp.float32)]
            + [pltpu.SemaphoreType.DMA] * 6
        ),
    )
    def k(idx_hbm, table_hbm, out_hbm, idx_v, r0, r1, sp, g0, g1, c0, c1, w0, w1):
        rows, gsem, csem, wsem = (r0, r1), (g0, g1), (c0, c1), (w0, w1)
        sid = lax.axis_index("s")
        wid = sid * nc + lax.axis_index("c")
        base = wid * per_w
        pltpu.sync_copy(idx_hbm.at[pl.ds(base, per_w)], idx_v)

        def idx_slice(i):
            return idx_v.at[pl.ds(pl.multiple_of(i * chunk, 8), chunk)]

        def gather_start(b, i):
            pltpu.async_copy(table_hbm.at[idx_slice(i)], rows[b], gsem[b])

        def gather_wait(b, i):
            pltpu.make_async_copy(table_hbm.at[idx_slice(i)], rows[b], gsem[b]).wait()

        def copy_start(b):
            pltpu.async_copy(rows[b], sp.at[sid, b], csem[b])

        def copy_wait(b):
            pltpu.make_async_copy(rows[b], sp.at[sid, b], csem[b]).wait()

        def wb_start(b, i):
            pltpu.async_copy(sp.at[sid, b], out_hbm.at[pl.ds(base + i * chunk, chunk)], wsem[b])

        def wb_wait(b):
            pltpu.make_async_copy(sp.at[sid, b], out_hbm.at[pl.ds(0, chunk)], wsem[b]).wait()

        for b in (0, 1):
            gather_start(b, b)

        for b in (0, 1):
            gather_wait(b, b)
            copy_start(b)
            copy_wait(b)
            wb_start(b, b)
            gather_start(b, b + 2)

        def group(g, carry):
            for b in (0, 1):
                i = 2 * g + b
                gather_wait(b, i)
                wb_wait(b)
                copy_start(b)
                copy_wait(b)
                wb_start(b, i)
                nxt = jnp.minimum(i + 2, n_chunks - 1)
                gather_start(b, nxt)
            return carry

        lax.fori_loop(1, n_groups, group, 0)

        for b in (0, 1):
            gather_wait(b, n_chunks - 1)
            wb_wait(b)

    return k(indices_flat, weight)


def kernel(indices, weight):
    flat = indices.reshape(-1).astype(jnp.int32)
    out = _embed(flat, weight)
    return out.reshape(BATCH, HIST, DIM)


# speedup vs baseline: 1.0099x; 1.0070x over previous
import functools

import jax
import jax.numpy as jnp
from jax import lax
from jax.experimental import pallas as pl
from jax.experimental.pallas import tpu as pltpu
from jax.experimental.pallas import tpu_sc as plsc

BATCH, HIST, DIM = 4096, 200, 128
TOTAL = BATCH * HIST


@functools.partial(jax.jit, static_argnames=())
def _embed(indices_flat, weight):
    info = plsc.get_sparse_core_info()
    nc, ns = info.num_cores, info.num_subcores
    nw = nc * ns
    per_w = TOTAL // nw
    chunk = 200
    n_chunks = per_w // chunk
    n_groups = n_chunks // 2

    mesh = plsc.VectorSubcoreMesh(core_axis_name="c", subcore_axis_name="s")

    @functools.partial(
        pl.kernel,
        mesh=mesh,
        out_type=jax.ShapeDtypeStruct((TOTAL, DIM), jnp.float32),
        scratch_types=(
            [pltpu.VMEM((per_w,), jnp.int32)]
            + [pltpu.VMEM((chunk, DIM), jnp.float32)] * 2
            + [pltpu.VMEM_SHARED((ns, 2, chunk, DIM), jnp.float32)]
            + [pltpu.SemaphoreType.DMA] * 6
        ),
    )
    def k(idx_hbm, table_hbm, out_hbm, idx_v, r0, r1, sp, g0, g1, c0, c1, w0, w1):
        rows, gsem, csem, wsem = (r0, r1), (g0, g1), (c0, c1), (w0, w1)
        sid = lax.axis_index("s")
        wid = sid * nc + lax.axis_index("c")
        base = wid * per_w
        pltpu.sync_copy(idx_hbm.at[pl.ds(base, per_w)], idx_v)

        def idx_slice(i):
            return idx_v.at[pl.ds(pl.multiple_of(i * chunk, 8), chunk)]

        def gather_start(b, i):
            pltpu.async_copy(table_hbm.at[idx_slice(i)], rows[b], gsem[b])

        def gather_wait(b, i):
            pltpu.make_async_copy(table_hbm.at[idx_slice(i)], rows[b], gsem[b]).wait()

        def copy_start(b):
            pltpu.async_copy(rows[b], sp.at[sid, b], csem[b])

        def copy_wait(b):
            pltpu.make_async_copy(rows[b], sp.at[sid, b], csem[b]).wait()

        def wb_start(b, i):
            pltpu.async_copy(sp.at[sid, b], out_hbm.at[pl.ds(base + i * chunk, chunk)], wsem[b])

        def wb_wait(b):
            pltpu.make_async_copy(sp.at[sid, b], out_hbm.at[pl.ds(0, chunk)], wsem[b]).wait()

        for b in (0, 1):
            gather_start(b, b)

        for b in (0, 1):
            gather_wait(b, b)
            copy_start(b)
            copy_wait(b)
            wb_start(b, b)
            gather_start(b, b + 2)

        def group(g, carry):
            for b in (0, 1):
                i = 2 * g + b
                gather_wait(b, i)
                wb_wait(b)
                copy_start(b)
                copy_wait(b)
                wb_start(b, i)
                nxt = jnp.minimum(i + 2, n_chunks - 1)
                gather_start(b, nxt)
            return carry

        lax.fori_loop(1, n_groups, group, 0)

        for b in (0, 1):
            gather_wait(b, n_chunks - 1)
            wb_wait(b)

    return k(indices_flat, weight)


def kernel(indices, weight):
    flat = indices.reshape(-1).astype(jnp.int32)
    out = _embed(flat, weight)
    return out.reshape(BATCH, HIST, DIM)
